# baseline (speedup 1.0000x reference)
import functools

import jax
import jax.numpy as jnp
from jax import lax
from jax.experimental import pallas as pl
from jax.experimental.pallas import tpu as pltpu
from jax.experimental.pallas import tpu_sc as plsc

VOCAB = 1000000
DIM = 32
N_NODES = 100000

_info = plsc.get_sparse_core_info()
_NC, _NS = _info.num_cores, _info.num_subcores
_NW = _NC * _NS

_B_PAD = ((N_NODES + 8 * _NW - 1) // (8 * _NW)) * (8 * _NW)
_B_PER_W = _B_PAD // _NW


def _make_gather():
    mesh = plsc.VectorSubcoreMesh(core_axis_name="c", subcore_axis_name="s")

    @functools.partial(
        pl.kernel,
        mesh=mesh,
        out_type=jax.ShapeDtypeStruct((_B_PAD, DIM), jnp.float32),
        scratch_types=[
            pltpu.VMEM((_B_PER_W,), jnp.int32),
            pltpu.VMEM((_B_PER_W, DIM), jnp.float32),
            pltpu.SemaphoreType.DMA,
        ],
        compiler_params=pltpu.CompilerParams(use_tc_tiling_on_sc=False),
    )
    def gather_kernel(table_hbm, idx_hbm, out_hbm, idx_v, rows_v, sem):
        wid = lax.axis_index("s") * _NC + lax.axis_index("c")
        base = wid * _B_PER_W
        pltpu.sync_copy(idx_hbm.at[pl.ds(base, _B_PER_W)], idx_v)
        pltpu.async_copy(table_hbm.at[idx_v], rows_v, sem).wait()
        pltpu.sync_copy(rows_v, out_hbm.at[pl.ds(base, _B_PER_W)])

    return gather_kernel


_gather = _make_gather()


def kernel(token_id, table):
    idx = jnp.zeros((_B_PAD,), jnp.int32).at[:N_NODES].set(token_id)
    out = _gather(table, idx)
    return out[:N_NODES]


# speedup vs baseline: 1.0596x; 1.0596x over previous
import functools

import jax
import jax.numpy as jnp
from jax import lax
from jax.experimental import pallas as pl
from jax.experimental.pallas import tpu as pltpu
from jax.experimental.pallas import tpu_sc as plsc

VOCAB = 1000000
DIM = 32
N_NODES = 100000

_info = plsc.get_sparse_core_info()
_NC, _NS = _info.num_cores, _info.num_subcores
_NW = _NC * _NS

_B_PER_W = ((N_NODES + _NW - 1) // _NW + 7) // 8 * 8
_B_LAST = N_NODES - (_NW - 1) * _B_PER_W


def _make_gather():
    mesh = plsc.VectorSubcoreMesh(core_axis_name="c", subcore_axis_name="s")

    @functools.partial(
        pl.kernel,
        mesh=mesh,
        out_type=jax.ShapeDtypeStruct((N_NODES, DIM), jnp.float32),
        scratch_types=[
            pltpu.VMEM((_B_PER_W,), jnp.int32),
            pltpu.VMEM((_B_PER_W, DIM), jnp.float32),
            pltpu.SemaphoreType.DMA,
        ],
        compiler_params=pltpu.CompilerParams(use_tc_tiling_on_sc=False),
    )
    def gather_kernel(table_hbm, idx_hbm, out_hbm, idx_v, rows_v, sem):
        wid = lax.axis_index("s") * _NC + lax.axis_index("c")
        base = wid * _B_PER_W

        @pl.when(wid < _NW - 1)
        def _full():
            pltpu.sync_copy(idx_hbm.at[pl.ds(base, _B_PER_W)], idx_v)
            pltpu.async_copy(table_hbm.at[idx_v], rows_v, sem).wait()
            pltpu.sync_copy(rows_v, out_hbm.at[pl.ds(base, _B_PER_W)])

        @pl.when(wid == _NW - 1)
        def _tail():
            pltpu.sync_copy(
                idx_hbm.at[pl.ds(base, _B_LAST)], idx_v.at[pl.ds(0, _B_LAST)]
            )
            pltpu.async_copy(
                table_hbm.at[idx_v.at[pl.ds(0, _B_LAST)]],
                rows_v.at[pl.ds(0, _B_LAST)],
                sem,
            ).wait()
            pltpu.sync_copy(
                rows_v.at[pl.ds(0, _B_LAST)], out_hbm.at[pl.ds(base, _B_LAST)]
            )

    return gather_kernel


_gather = _make_gather()


def kernel(token_id, table):
    return _gather(table, token_id)
